# baseline (speedup 1.0000x reference)
import jax
import jax.numpy as jnp
from jax.experimental import pallas as pl


def _mlp_kernel(u_ref, w1_ref, b1_ref, w2_ref, b2_ref, o_ref):
    h = jnp.dot(u_ref[:], w1_ref[:], preferred_element_type=jnp.float32, precision=jax.lax.Precision.DEFAULT)
    h = h + b1_ref[:]
    h = jnp.where(h >= 0, h, 0.2 * h)
    o = jnp.dot(h, w2_ref[:], preferred_element_type=jnp.float32, precision=jax.lax.Precision.DEFAULT)
    o_ref[:] = o + b2_ref[:]


def kernel(u_st, W1, b1, W2, b2):
    n, d = u_st.shape
    hdim = W1.shape[0]
    block = 10000
    nbuf = 2
    grid = (n // block,)
    return pl.pallas_call(
        _mlp_kernel,
        grid=grid,
        in_specs=[
            pl.BlockSpec((block, d), lambda i: (i, 0),
                         pipeline_mode=pl.Buffered(buffer_count=nbuf)),
            pl.BlockSpec((d, hdim), lambda i: (0, 0)),
            pl.BlockSpec((1, hdim), lambda i: (0, 0)),
            pl.BlockSpec((hdim, d), lambda i: (0, 0)),
            pl.BlockSpec((1, d), lambda i: (0, 0)),
        ],
        out_specs=pl.BlockSpec((block, d), lambda i: (i, 0),
                               pipeline_mode=pl.Buffered(buffer_count=nbuf)),
        out_shape=jax.ShapeDtypeStruct((n, d), jnp.float32),
    )(u_st, W1.T, b1.reshape(1, hdim), W2.T, b2.reshape(1, d))


# speedup vs baseline: 1.0019x; 1.0019x over previous
import jax
import jax.numpy as jnp
from jax.experimental import pallas as pl
from jax.experimental.pallas import tpu as pltpu


def _mlp_kernel(u_ref, w1_ref, b1_ref, w2_ref, b2_ref, o_ref):
    h = jnp.dot(u_ref[:], w1_ref[:], preferred_element_type=jnp.float32, precision=jax.lax.Precision.DEFAULT)
    h = h + b1_ref[:]
    h = jnp.where(h >= 0, h, 0.2 * h)
    o = jnp.dot(h, w2_ref[:], preferred_element_type=jnp.float32, precision=jax.lax.Precision.DEFAULT)
    o_ref[:] = o + b2_ref[:]


def kernel(u_st, W1, b1, W2, b2):
    n, d = u_st.shape
    hdim = W1.shape[0]
    block = 10000
    nbuf = 2
    grid = (n // block,)
    return pl.pallas_call(
        _mlp_kernel,
        grid=grid,
        in_specs=[
            pl.BlockSpec((block, d), lambda i: (i, 0),
                         pipeline_mode=pl.Buffered(buffer_count=nbuf)),
            pl.BlockSpec((d, hdim), lambda i: (0, 0)),
            pl.BlockSpec((1, hdim), lambda i: (0, 0)),
            pl.BlockSpec((hdim, d), lambda i: (0, 0)),
            pl.BlockSpec((1, d), lambda i: (0, 0)),
        ],
        out_specs=pl.BlockSpec((block, d), lambda i: (i, 0),
                               pipeline_mode=pl.Buffered(buffer_count=nbuf)),
        out_shape=jax.ShapeDtypeStruct((n, d), jnp.float32),
        compiler_params=pltpu.CompilerParams(
            dimension_semantics=("parallel",),
        ),
    )(u_st, W1.T, b1.reshape(1, hdim), W2.T, b2.reshape(1, d))
